# baseline (speedup 1.0000x reference)
import functools

import jax
import jax.numpy as jnp
from jax.experimental import pallas as pl

N, F, H, O = 10000, 128, 128, 128
BR = 200


def _xw_body(x_ref, w_ref, o_ref):
    o_ref[...] = jnp.dot(x_ref[...], w_ref[...],
                         preferred_element_type=jnp.float32)


def _main_body(adj_ref, xw_ref, b0_ref, w1_ref, b1_ref, w2_ref, b2_ref,
               out_ref):
    h = jnp.dot(adj_ref[...], xw_ref[...],
                preferred_element_type=jnp.float32)
    h = jnp.maximum(h + b0_ref[...], 0.0)
    h = jnp.dot(h, w1_ref[...], preferred_element_type=jnp.float32)
    h = jnp.maximum(h + b1_ref[...], 0.0)
    out_ref[...] = jnp.dot(h, w2_ref[...],
                           preferred_element_type=jnp.float32) + b2_ref[...]


@functools.partial(jax.jit, static_argnames=())
def kernel(adj, initial_features, gcn_W, gcn_b, lin1_W, lin1_b,
           bn_gamma, bn_beta, bn_mean, bn_var, lin2_W, lin2_b):
    scale = bn_gamma * jax.lax.rsqrt(bn_var + 1e-5)
    w1 = lin1_W * scale[None, :]
    b1 = (lin1_b - bn_mean) * scale + bn_beta

    xw = pl.pallas_call(
        _xw_body,
        out_shape=jax.ShapeDtypeStruct((N, H), jnp.float32),
    )(initial_features, gcn_W)

    b0_2d = gcn_b.reshape(1, H)
    b1_2d = b1.reshape(1, H)
    b2_2d = lin2_b.reshape(1, O)

    const = lambda shape: pl.BlockSpec(shape, lambda i: (0, 0))
    out = pl.pallas_call(
        _main_body,
        grid=(N // BR,),
        in_specs=[
            pl.BlockSpec((BR, N), lambda i: (i, 0)),
            const((N, H)),
            const((1, H)),
            const((H, H)),
            const((1, H)),
            const((H, O)),
            const((1, O)),
        ],
        out_specs=pl.BlockSpec((BR, O), lambda i: (i, 0)),
        out_shape=jax.ShapeDtypeStruct((N, O), jnp.float32),
    )(adj, xw, b0_2d, w1, b1_2d, lin2_W, b2_2d)
    return out


# speedup vs baseline: 1.0931x; 1.0931x over previous
import functools

import jax
import jax.numpy as jnp
from jax.experimental import pallas as pl
from jax.experimental.pallas import tpu as pltpu

N, F, H, O = 10000, 128, 128, 128
BR = 400


def _body(adj_ref, x_ref, gw_ref, b0_ref, w1_ref, b1_ref, w2_ref, b2_ref,
          out_ref, xw_ref):
    @pl.when(pl.program_id(0) == 0)
    def _compute_xw():
        xw_ref[...] = jnp.dot(x_ref[...], gw_ref[...],
                              preferred_element_type=jnp.float32)

    h = jnp.dot(adj_ref[...], xw_ref[...],
                preferred_element_type=jnp.float32)
    h = jnp.maximum(h + b0_ref[...], 0.0)
    h = jnp.dot(h, w1_ref[...], preferred_element_type=jnp.float32)
    h = jnp.maximum(h + b1_ref[...], 0.0)
    out_ref[...] = jnp.dot(h, w2_ref[...],
                           preferred_element_type=jnp.float32) + b2_ref[...]


@functools.partial(jax.jit, static_argnames=())
def kernel(adj, initial_features, gcn_W, gcn_b, lin1_W, lin1_b,
           bn_gamma, bn_beta, bn_mean, bn_var, lin2_W, lin2_b):
    scale = bn_gamma * jax.lax.rsqrt(bn_var + 1e-5)
    w1 = lin1_W * scale[None, :]
    b1 = (lin1_b - bn_mean) * scale + bn_beta

    b0_2d = gcn_b.reshape(1, H)
    b1_2d = b1.reshape(1, H)
    b2_2d = lin2_b.reshape(1, O)

    const = lambda shape: pl.BlockSpec(shape, lambda i: (0, 0))
    out = pl.pallas_call(
        _body,
        grid=(N // BR,),
        in_specs=[
            pl.BlockSpec((BR, N), lambda i: (i, 0)),
            const((N, F)),
            const((F, H)),
            const((1, H)),
            const((H, H)),
            const((1, H)),
            const((H, O)),
            const((1, O)),
        ],
        out_specs=pl.BlockSpec((BR, O), lambda i: (i, 0)),
        out_shape=jax.ShapeDtypeStruct((N, O), jnp.float32),
        scratch_shapes=[pltpu.VMEM((N, H), jnp.float32)],
    )(adj, initial_features, gcn_W, b0_2d, w1, b1_2d, lin2_W, b2_2d)
    return out
